# baseline (speedup 1.0000x reference)
import functools

import jax
import jax.numpy as jnp
from jax import lax
from jax.experimental import pallas as pl
from jax.experimental.pallas import tpu as pltpu
from jax.experimental.pallas import tpu_sc as plsc

_D = 64
_BATCH = 16384
_HIST = 50
_B_TOTAL = _BATCH * _HIST
_NC = 2
_NS = 16
_NW = _NC * _NS
_BW = 128
_NTB = _BATCH // _BW
_NBLK = _HIST * _NTB
_BLK_PER_W = _NBLK // _NW
_QROWS = _B_TOTAL * _D // 128
_TW = _BW + 1


def _make_gather(mesh):
    @functools.partial(
        pl.kernel,
        mesh=mesh,
        out_type=jax.ShapeDtypeStruct((_QROWS, 128), jnp.float32),
        compiler_params=pltpu.CompilerParams(
            use_tc_tiling_on_sc=False, needs_layout_passes=False),
        scratch_types=[
            pltpu.VMEM((_BLK_PER_W, _BW), jnp.int32),
            pltpu.VMEM((_BW, _D), jnp.float32),
            pltpu.VMEM((_BW, _D), jnp.float32),
            pltpu.VMEM((_BW, _D), jnp.float32),
            pltpu.VMEM((_BW, _D), jnp.float32),
            pltpu.VMEM((_D, _TW), jnp.float32),
            pltpu.VMEM((_D, _TW), jnp.float32),
            pltpu.VMEM((_D, _TW), jnp.float32),
            pltpu.VMEM((_D, _TW), jnp.float32),
            pltpu.SemaphoreType.DMA,
            pltpu.SemaphoreType.DMA,
            pltpu.SemaphoreType.DMA,
            pltpu.SemaphoreType.DMA,
            pltpu.SemaphoreType.DMA,
            pltpu.SemaphoreType.DMA,
            pltpu.SemaphoreType.DMA,
            pltpu.SemaphoreType.DMA,
        ],
    )
    def gather(idx_hbm, table_hbm, out_hbm, idx_v,
               rows0, rows1, rows2, rows3, t0, t1, t2, t3,
               g0, g1, g2, g3, w0, w1, w2, w3):
        rows_b = (rows0, rows1, rows2, rows3)
        t_b = (t0, t1, t2, t3)
        g_b = (g0, g1, g2, g3)
        w_b = (w0, w1, w2, w3)
        wid = lax.axis_index("s") * _NC + lax.axis_index("c")
        blk0 = wid * _BLK_PER_W
        pltpu.sync_copy(idx_hbm.at[pl.ds(blk0, _BLK_PER_W)], idx_v)

        lane = lax.iota(jnp.int32, 16)

        def fire(i, slot, sem):
            pltpu.async_copy(table_hbm.at[idx_v.at[i]], rows_b[slot], sem)

        def gwait(i, slot, sem):
            pltpu.make_async_copy(
                table_hbm.at[idx_v.at[i]], rows_b[slot], sem).wait()

        def transpose(slot):
            src = rows_b[slot]
            dst = t_b[slot]

            @plsc.parallel_loop(0, _BW, step=2, unroll=4)
            def _(j):
                js0 = jnp.full((16,), 0, jnp.int32) + j
                js1 = js0 + 1
                vs = []
                for k in range(_D // 16):
                    vs.append(src[j, pl.ds(16 * k, 16)])
                for k in range(_D // 16):
                    vs.append(src[j + 1, pl.ds(16 * k, 16)])
                for k in range(_D // 16):
                    plsc.store_scatter(dst, [lane + (16 * k), js0], vs[k])
                for k in range(_D // 16):
                    plsc.store_scatter(dst, [lane + (16 * k), js1], vs[4 + k])

        def wstart(i, slot, sem):
            blk = blk0 + i
            h = blk // _NTB
            tb = blk % _NTB
            for a in range(_D // 8):
                pltpu.async_copy(
                    t_b[slot].at[pl.ds(a * 8, 8), pl.ds(0, _BW)],
                    out_hbm.at[pl.ds(h * 8192 + a * 1024 + tb * 8, 8)],
                    sem)

        def wwait(i, slot, sem):
            blk = blk0 + i
            h = blk // _NTB
            tb = blk % _NTB
            for a in range(_D // 8):
                pltpu.make_async_copy(
                    t_b[slot].at[pl.ds(a * 8, 8), pl.ds(0, _BW)],
                    out_hbm.at[pl.ds(h * 8192 + a * 1024 + tb * 8, 8)],
                    sem).wait()

        for s in range(4):
            fire(s, s, g_b[s])

        n_groups = _BLK_PER_W // 4

        @pl.loop(0, n_groups)
        def group(p):
            for s in range(4):
                i = p * 4 + s
                gwait(i, s, g_b[s])

                @pl.when(p != 0)
                def _():
                    wwait(i - 4, s, w_b[s])

                transpose(s)
                wstart(i, s, w_b[s])

                @pl.when(p != n_groups - 1)
                def _():
                    fire(i + 4, s, g_b[s])

        for s in range(4):
            wwait(_BLK_PER_W - 4 + s, s, w_b[s])

    return gather


_V = 1000000
_CB = 2048


def _transpose_table(table_t):
    def body(tin_ref, tout_ref):
        x = tin_ref[...]
        eye = (lax.broadcasted_iota(jnp.int32, (_D, _D), 0)
               == lax.broadcasted_iota(jnp.int32, (_D, _D), 1)
               ).astype(jnp.float32)
        y = lax.dot_general(x, eye, (((0,), (0,)), ((), ())),
                            preferred_element_type=jnp.float32)
        z = y.reshape(_CB // 2, 2, _D)
        tout_ref[:, 0:_D] = z[:, 0, :]
        tout_ref[:, _D:128] = z[:, 1, :]

    grid = (_V + _CB - 1) // _CB
    return pl.pallas_call(
        body,
        grid=(grid,),
        in_specs=[pl.BlockSpec((_D, _CB), lambda c: (0, c))],
        out_specs=pl.BlockSpec((_CB // 2, 128), lambda c: (c, 0)),
        out_shape=jax.ShapeDtypeStruct((_V // 2, 128), jnp.float32),
    )(table_t)


def kernel(inp, table):
    mesh = plsc.VectorSubcoreMesh(core_axis_name="c", subcore_axis_name="s")
    idx_blocks = jnp.transpose(inp.astype(jnp.int32)).reshape(_NBLK, _BW)
    table_lin = _transpose_table(jnp.transpose(table)).reshape(_V, _D)
    out_lin = _make_gather(mesh)(idx_blocks, table_lin)
    out = out_lin.reshape(_HIST, _D // 8, _NTB, 8, _BW)
    out = out.transpose(2, 4, 0, 1, 3).reshape(_BATCH, _HIST, _D)
    return out


# speedup vs baseline: 1.3018x; 1.3018x over previous
import functools

import jax
import jax.numpy as jnp
from jax import lax
from jax.experimental import pallas as pl
from jax.experimental.pallas import tpu as pltpu
from jax.experimental.pallas import tpu_sc as plsc

_D = 64
_BATCH = 16384
_HIST = 50
_B_TOTAL = _BATCH * _HIST
_NC = 2
_NS = 16
_NW = _NC * _NS
_BW = 128
_NTB = _BATCH // _BW
_NBLK = _HIST * _NTB
_BLK_PER_W = _NBLK // _NW
_QROWS = _B_TOTAL * _D // 128
_TW = _BW + 1


def _make_gather(mesh):
    @functools.partial(
        pl.kernel,
        mesh=mesh,
        out_type=jax.ShapeDtypeStruct((_QROWS, 128), jnp.float32),
        compiler_params=pltpu.CompilerParams(
            use_tc_tiling_on_sc=False, needs_layout_passes=False),
        scratch_types=[
            pltpu.VMEM((_BLK_PER_W, _BW), jnp.int32),
            pltpu.VMEM((_BW, _D), jnp.float32),
            pltpu.VMEM((_BW, _D), jnp.float32),
            pltpu.VMEM((_BW, _D), jnp.float32),
            pltpu.VMEM((_BW, _D), jnp.float32),
            pltpu.VMEM((_D, _TW), jnp.float32),
            pltpu.VMEM((_D, _TW), jnp.float32),
            pltpu.VMEM((_D, _TW), jnp.float32),
            pltpu.VMEM((_D, _TW), jnp.float32),
            pltpu.SemaphoreType.DMA,
            pltpu.SemaphoreType.DMA,
            pltpu.SemaphoreType.DMA,
            pltpu.SemaphoreType.DMA,
            pltpu.SemaphoreType.DMA,
            pltpu.SemaphoreType.DMA,
            pltpu.SemaphoreType.DMA,
            pltpu.SemaphoreType.DMA,
        ],
    )
    def gather(idx_hbm, table_hbm, out_hbm, idx_v,
               rows0, rows1, rows2, rows3, t0, t1, t2, t3,
               g0, g1, g2, g3, w0, w1, w2, w3):
        rows_b = (rows0, rows1, rows2, rows3)
        t_b = (t0, t1, t2, t3)
        g_b = (g0, g1, g2, g3)
        w_b = (w0, w1, w2, w3)
        wid = lax.axis_index("s") * _NC + lax.axis_index("c")
        blk0 = wid * _BLK_PER_W
        pltpu.sync_copy(idx_hbm.at[pl.ds(blk0, _BLK_PER_W)], idx_v)

        lane = lax.iota(jnp.int32, 16)

        def fire(i, slot, sem):
            pltpu.async_copy(table_hbm.at[idx_v.at[i]], rows_b[slot], sem)

        def gwait(i, slot, sem):
            pltpu.make_async_copy(
                table_hbm.at[idx_v.at[i]], rows_b[slot], sem).wait()

        def transpose(slot):
            src = rows_b[slot]
            dst = t_b[slot]

            @plsc.parallel_loop(0, _BW, step=2, unroll=4)
            def _(j):
                js0 = jnp.full((16,), 0, jnp.int32) + j
                js1 = js0 + 1
                vs = []
                for k in range(_D // 16):
                    vs.append(src[j, pl.ds(16 * k, 16)])
                for k in range(_D // 16):
                    vs.append(src[j + 1, pl.ds(16 * k, 16)])
                for k in range(_D // 16):
                    plsc.store_scatter(dst, [lane + (16 * k), js0], vs[k])
                for k in range(_D // 16):
                    plsc.store_scatter(dst, [lane + (16 * k), js1], vs[4 + k])

        def wstart(i, slot, sem):
            blk = blk0 + i
            h = blk // _NTB
            tb = blk % _NTB
            for a in range(_D // 8):
                pltpu.async_copy(
                    t_b[slot].at[pl.ds(a * 8, 8), pl.ds(0, _BW)],
                    out_hbm.at[pl.ds(h * 8192 + a * 1024 + tb * 8, 8)],
                    sem)

        def wwait(i, slot, sem):
            blk = blk0 + i
            h = blk // _NTB
            tb = blk % _NTB
            for a in range(_D // 8):
                pltpu.make_async_copy(
                    t_b[slot].at[pl.ds(a * 8, 8), pl.ds(0, _BW)],
                    out_hbm.at[pl.ds(h * 8192 + a * 1024 + tb * 8, 8)],
                    sem).wait()

        for s in range(4):
            fire(s, s, g_b[s])

        n_groups = _BLK_PER_W // 4

        @pl.loop(0, n_groups)
        def group(p):
            for s in range(4):
                i = p * 4 + s
                gwait(i, s, g_b[s])

                @pl.when(p != 0)
                def _():
                    wwait(i - 4, s, w_b[s])

                transpose(s)
                wstart(i, s, w_b[s])

                @pl.when(p != n_groups - 1)
                def _():
                    fire(i + 4, s, g_b[s])

        for s in range(4):
            wwait(_BLK_PER_W - 4 + s, s, w_b[s])

    return gather


_V = 1000000
_CB = 8192


def _transpose_table(table_t):
    def body(tin_ref, tout_ref):
        y = jnp.transpose(tin_ref[...])
        z = y.reshape(_CB // 2, 2, _D)
        tout_ref[:, 0:_D] = z[:, 0, :]
        tout_ref[:, _D:128] = z[:, 1, :]

    grid = (_V + _CB - 1) // _CB
    return pl.pallas_call(
        body,
        grid=(grid,),
        in_specs=[pl.BlockSpec((_D, _CB), lambda c: (0, c))],
        out_specs=pl.BlockSpec((_CB // 2, 128), lambda c: (c, 0)),
        out_shape=jax.ShapeDtypeStruct((_V // 2, 128), jnp.float32),
    )(table_t)


def kernel(inp, table):
    mesh = plsc.VectorSubcoreMesh(core_axis_name="c", subcore_axis_name="s")
    idx_blocks = jnp.transpose(inp.astype(jnp.int32)).reshape(_NBLK, _BW)
    table_lin = _transpose_table(jnp.transpose(table)).reshape(_V, _D)
    out_lin = _make_gather(mesh)(idx_blocks, table_lin)
    out = out_lin.reshape(_HIST, _D // 8, _NTB, 8, _BW)
    out = out.transpose(2, 4, 0, 1, 3).reshape(_BATCH, _HIST, _D)
    return out


# speedup vs baseline: 1.3179x; 1.0124x over previous
import functools

import jax
import jax.numpy as jnp
from jax import lax
from jax.experimental import pallas as pl
from jax.experimental.pallas import tpu as pltpu
from jax.experimental.pallas import tpu_sc as plsc

_D = 64
_BATCH = 16384
_HIST = 50
_B_TOTAL = _BATCH * _HIST
_NC = 2
_NS = 16
_NW = _NC * _NS
_BW = 128
_NTB = _BATCH // _BW
_NBLK = _HIST * _NTB
_BLK_PER_W = _NBLK // _NW
_QROWS = _B_TOTAL * _D // 128
_TW = _BW + 1


def _make_gather(mesh):
    @functools.partial(
        pl.kernel,
        mesh=mesh,
        out_type=jax.ShapeDtypeStruct((_QROWS, 128), jnp.float32),
        compiler_params=pltpu.CompilerParams(
            use_tc_tiling_on_sc=False, needs_layout_passes=False),
        scratch_types=[
            pltpu.VMEM((_BLK_PER_W, _BW), jnp.int32),
            pltpu.VMEM((_BW, _D), jnp.float32),
            pltpu.VMEM((_BW, _D), jnp.float32),
            pltpu.VMEM((_BW, _D), jnp.float32),
            pltpu.VMEM((_BW, _D), jnp.float32),
            pltpu.VMEM((_D, _TW), jnp.float32),
            pltpu.VMEM((_D, _TW), jnp.float32),
            pltpu.VMEM((_D, _TW), jnp.float32),
            pltpu.VMEM((_D, _TW), jnp.float32),
            pltpu.SemaphoreType.DMA,
            pltpu.SemaphoreType.DMA,
            pltpu.SemaphoreType.DMA,
            pltpu.SemaphoreType.DMA,
            pltpu.SemaphoreType.DMA,
            pltpu.SemaphoreType.DMA,
            pltpu.SemaphoreType.DMA,
            pltpu.SemaphoreType.DMA,
        ],
    )
    def gather(idx_hbm, table_hbm, out_hbm, idx_v,
               rows0, rows1, rows2, rows3, t0, t1, t2, t3,
               g0, g1, g2, g3, w0, w1, w2, w3):
        rows_b = (rows0, rows1, rows2, rows3)
        t_b = (t0, t1, t2, t3)
        g_b = (g0, g1, g2, g3)
        w_b = (w0, w1, w2, w3)
        wid = lax.axis_index("s") * _NC + lax.axis_index("c")
        blk0 = wid * _BLK_PER_W
        pltpu.sync_copy(idx_hbm.at[pl.ds(blk0, _BLK_PER_W)], idx_v)

        lane = lax.iota(jnp.int32, 16)

        def fire(i, slot, sem):
            pltpu.async_copy(table_hbm.at[idx_v.at[i]], rows_b[slot], sem)

        def gwait(i, slot, sem):
            pltpu.make_async_copy(
                table_hbm.at[idx_v.at[i]], rows_b[slot], sem).wait()

        def transpose(slot):
            src = rows_b[slot]
            dst = t_b[slot]

            @plsc.parallel_loop(0, _BW, step=2, unroll=4)
            def _(j):
                js0 = jnp.full((16,), 0, jnp.int32) + j
                js1 = js0 + 1
                vs = []
                for k in range(_D // 16):
                    vs.append(src[j, pl.ds(16 * k, 16)])
                for k in range(_D // 16):
                    vs.append(src[j + 1, pl.ds(16 * k, 16)])
                for k in range(_D // 16):
                    plsc.store_scatter(dst, [lane + (16 * k), js0], vs[k])
                for k in range(_D // 16):
                    plsc.store_scatter(dst, [lane + (16 * k), js1], vs[4 + k])

        def wstart(i, slot, sem):
            blk = blk0 + i
            h = blk // _NTB
            tb = blk % _NTB
            for a in range(_D // 8):
                pltpu.async_copy(
                    t_b[slot].at[pl.ds(a * 8, 8), pl.ds(0, _BW)],
                    out_hbm.at[pl.ds(h * 8192 + a * 1024 + tb * 8, 8)],
                    sem)

        def wwait(i, slot, sem):
            blk = blk0 + i
            h = blk // _NTB
            tb = blk % _NTB
            for a in range(_D // 8):
                pltpu.make_async_copy(
                    t_b[slot].at[pl.ds(a * 8, 8), pl.ds(0, _BW)],
                    out_hbm.at[pl.ds(h * 8192 + a * 1024 + tb * 8, 8)],
                    sem).wait()

        for s in range(4):
            fire(s, s, g_b[s])

        n_groups = _BLK_PER_W // 4

        @pl.loop(0, n_groups)
        def group(p):
            for s in range(4):
                i = p * 4 + s
                gwait(i, s, g_b[s])

                @pl.when(p != 0)
                def _():
                    wwait(i - 4, s, w_b[s])

                transpose(s)
                wstart(i, s, w_b[s])

                @pl.when(p != n_groups - 1)
                def _():
                    fire(i + 4, s, g_b[s])

        for s in range(4):
            wwait(_BLK_PER_W - 4 + s, s, w_b[s])

    return gather


_V = 1000000
_CB = 16384


def _transpose_table(table_t):
    def body(tin_ref, tout_ref):
        y = jnp.transpose(tin_ref[...])
        z = y.reshape(_CB // 2, 2, _D)
        tout_ref[:, 0:_D] = z[:, 0, :]
        tout_ref[:, _D:128] = z[:, 1, :]

    grid = (_V + _CB - 1) // _CB
    return pl.pallas_call(
        body,
        grid=(grid,),
        in_specs=[pl.BlockSpec((_D, _CB), lambda c: (0, c))],
        out_specs=pl.BlockSpec((_CB // 2, 128), lambda c: (c, 0)),
        out_shape=jax.ShapeDtypeStruct((_V // 2, 128), jnp.float32),
    )(table_t)


def kernel(inp, table):
    mesh = plsc.VectorSubcoreMesh(core_axis_name="c", subcore_axis_name="s")
    idx_blocks = jnp.transpose(inp.astype(jnp.int32)).reshape(_NBLK, _BW)
    table_lin = _transpose_table(jnp.transpose(table)).reshape(_V, _D)
    out_lin = _make_gather(mesh)(idx_blocks, table_lin)
    out = out_lin.reshape(_HIST, _D // 8, _NTB, 8, _BW)
    out = out.transpose(2, 4, 0, 1, 3).reshape(_BATCH, _HIST, _D)
    return out
